# Initial kernel scaffold; baseline (speedup 1.0000x reference)
import jax
import jax.numpy as jnp
from jax.experimental import pallas as pl


def kernel(x, edge_index, W1, b1, W2, b2):
    raise NotImplementedError("write your pallas kernel here")


# speedup vs baseline: 9.8949x; 9.8949x over previous
import functools

import jax
import jax.numpy as jnp
from jax import lax
from jax.experimental import pallas as pl
from jax.experimental.pallas import tpu as pltpu
from jax.experimental.pallas import tpu_sc as plsc

N = 10000
E = 320000
IN_CH = 128
H1 = 256
H2 = 128
NC, NS = 2, 16
NW = NC * NS
EPT = E // NW
CH = 80
NCHUNK = EPT // CH
ACC_N = 10240
RPT = ACC_N // NS
DEGW = 128
SLOPE = (1.0 / 8 + 1.0 / 3) / 2.0

_MESH = plsc.VectorSubcoreMesh(
    core_axis_name="c", subcore_axis_name="s", num_cores=NC, num_subcores=NS
)


@functools.partial(
    pl.kernel,
    out_type=jax.ShapeDtypeStruct((NC, ACC_N, DEGW), jnp.float32),
    mesh=_MESH,
    scratch_types=[
        pltpu.VMEM((CH,), jnp.int32),
        pltpu.VMEM((CH, DEGW), jnp.float32),
        pltpu.VMEM_SHARED((ACC_N, DEGW), jnp.float32),
    ],
)
def _deg_kernel(dst_hbm, ones_hbm, zero_hbm, out_hbm, didx, ones_v, acc):
    c = lax.axis_index("c")
    s = lax.axis_index("s")
    base0 = (c * NS + s) * EPT
    r0 = s * RPT
    pltpu.sync_copy(ones_hbm, ones_v)
    pltpu.sync_copy(zero_hbm.at[pl.ds(r0, RPT)], acc.at[pl.ds(r0, RPT)])
    plsc.subcore_barrier()

    def body(i, carry):
        b = base0 + i * CH
        pltpu.sync_copy(dst_hbm.at[pl.ds(b, CH)], didx)
        pltpu.sync_copy(ones_v, acc.at[didx], add=True)
        return carry

    lax.fori_loop(0, NCHUNK, body, 0)
    plsc.subcore_barrier()
    pltpu.sync_copy(acc.at[pl.ds(r0, RPT)], out_hbm.at[c, pl.ds(r0, RPT)])


@functools.partial(
    pl.kernel,
    out_type=jax.ShapeDtypeStruct((NC, ACC_N, H2), jnp.float32),
    mesh=_MESH,
    scratch_types=[
        pltpu.VMEM((CH,), jnp.int32),
        pltpu.VMEM((CH,), jnp.int32),
        pltpu.VMEM((CH, H2), jnp.float32),
        pltpu.VMEM_SHARED((ACC_N, H2), jnp.float32),
        pltpu.SemaphoreType.DMA,
    ],
)
def _spmm_kernel(table_hbm, src_hbm, dst_hbm, zero_hbm, out_hbm,
                 sidx, didx, rows, acc, sem):
    c = lax.axis_index("c")
    s = lax.axis_index("s")
    base0 = (c * NS + s) * EPT
    r0 = s * RPT
    pltpu.sync_copy(zero_hbm.at[pl.ds(r0, RPT)], acc.at[pl.ds(r0, RPT)])
    plsc.subcore_barrier()

    def body(i, carry):
        b = base0 + i * CH
        pltpu.sync_copy(src_hbm.at[pl.ds(b, CH)], sidx)
        pltpu.sync_copy(dst_hbm.at[pl.ds(b, CH)], didx)
        pltpu.async_copy(table_hbm.at[sidx], rows, sem).wait()
        pltpu.sync_copy(rows, acc.at[didx], add=True)
        return carry

    lax.fori_loop(0, NCHUNK, body, 0)
    plsc.subcore_barrier()
    pltpu.sync_copy(acc.at[pl.ds(r0, RPT)], out_hbm.at[c, pl.ds(r0, RPT)])


_RB = 2000


def _rsqrt_deg(degp_ref):
    deg = degp_ref[0, :, 0:1] + degp_ref[1, :, 0:1] + 1.0
    return lax.rsqrt(deg)


def _tc1_body(x_ref, w1_ref, degp_ref, h1a_ref, h1b_ref):
    xh = jnp.dot(x_ref[...], w1_ref[...], preferred_element_type=jnp.float32)
    d = _rsqrt_deg(degp_ref)
    h = xh * d
    h1a_ref[...] = h[:, :H2]
    h1b_ref[...] = h[:, H2:]


def _tc2_body(a1a_ref, a1b_ref, h1a_ref, h1b_ref, degp_ref,
              w2a_ref, w2b_ref, b1_ref, h2_ref):
    d = _rsqrt_deg(degp_ref)
    ua = d * (a1a_ref[0] + a1a_ref[1] + h1a_ref[...]) + b1_ref[:, :H2]
    ub = d * (a1b_ref[0] + a1b_ref[1] + h1b_ref[...]) + b1_ref[:, H2:]
    ra = jnp.where(ua >= 0, ua, ua * SLOPE)
    rb = jnp.where(ub >= 0, ub, ub * SLOPE)
    xh2 = (jnp.dot(ra, w2a_ref[...], preferred_element_type=jnp.float32)
           + jnp.dot(rb, w2b_ref[...], preferred_element_type=jnp.float32))
    h2_ref[...] = xh2 * d


def _tc3_body(a2_ref, h2_ref, degp_ref, b2_ref, z_ref):
    d = _rsqrt_deg(degp_ref)
    z_ref[...] = d * (a2_ref[0] + a2_ref[1] + h2_ref[...]) + b2_ref[...]


def _row_spec(w):
    return pl.BlockSpec((_RB, w), lambda i: (i, 0))


def _part_spec(w):
    return pl.BlockSpec((NC, _RB, w), lambda i: (0, i, 0))


_DEG_SPEC = pl.BlockSpec((NC, _RB, DEGW), lambda i: (0, i, 0))
_GRID = (N // _RB,)

_tc1 = pl.pallas_call(
    _tc1_body,
    grid=_GRID,
    in_specs=[
        _row_spec(IN_CH),
        pl.BlockSpec((IN_CH, H1), lambda i: (0, 0)),
        _DEG_SPEC,
    ],
    out_specs=[_row_spec(H2), _row_spec(H2)],
    out_shape=[
        jax.ShapeDtypeStruct((N, H2), jnp.float32),
        jax.ShapeDtypeStruct((N, H2), jnp.float32),
    ],
)

_tc2 = pl.pallas_call(
    _tc2_body,
    grid=_GRID,
    in_specs=[
        _part_spec(H2),
        _part_spec(H2),
        _row_spec(H2),
        _row_spec(H2),
        _DEG_SPEC,
        pl.BlockSpec((H2, H2), lambda i: (0, 0)),
        pl.BlockSpec((H2, H2), lambda i: (0, 0)),
        pl.BlockSpec((1, H1), lambda i: (0, 0)),
    ],
    out_specs=_row_spec(H2),
    out_shape=jax.ShapeDtypeStruct((N, H2), jnp.float32),
)

_tc3 = pl.pallas_call(
    _tc3_body,
    grid=_GRID,
    in_specs=[
        _part_spec(H2),
        _row_spec(H2),
        _DEG_SPEC,
        pl.BlockSpec((1, H2), lambda i: (0, 0)),
    ],
    out_specs=_row_spec(H2),
    out_shape=jax.ShapeDtypeStruct((N, H2), jnp.float32),
)


def kernel(x, edge_index, W1, b1, W2, b2):
    src = edge_index[0].astype(jnp.int32)
    dst = edge_index[1].astype(jnp.int32)
    ones16 = jnp.ones((CH, DEGW), jnp.float32)
    z16 = jnp.zeros((ACC_N, DEGW), jnp.float32)
    z128 = jnp.zeros((ACC_N, H2), jnp.float32)

    degp = _deg_kernel(dst, ones16, z16)
    h1a, h1b = _tc1(x, W1, degp)
    agg1a = _spmm_kernel(h1a, src, dst, z128)
    agg1b = _spmm_kernel(h1b, src, dst, z128)
    h2 = _tc2(agg1a, agg1b, h1a, h1b, degp,
              W2[:H2], W2[H2:], b1.reshape(1, H1))
    agg2 = _spmm_kernel(h2, src, dst, z128)
    z = _tc3(agg2, h2, degp, b2.reshape(1, H2))
    return z
